# baseline (speedup 1.0000x reference)
import jax
import jax.numpy as jnp
from jax import lax
from jax.experimental import pallas as pl
from jax.experimental.pallas import tpu as pltpu
from jax.experimental.pallas import tpu_sc as plsc

B = 1024
L = 200
H = 768
VOCAB_ROWS = 30522
EPS = 1e-12

NC = 2
NS = 16
NW = NC * NS
TOK = B * L
TOK_W = TOK // NW
GCHUNK = 64
NG = TOK_W // GCHUNK
NSLOT = 4

RB = 8


def _gather_body(ids_hbm, word_hbm, g_hbm, idbuf, slots, gsem, wbsem):
    wid = lax.axis_index("s") * NC + lax.axis_index("c")
    w0 = wid * TOK_W
    pltpu.sync_copy(ids_hbm.at[pl.ds(w0, TOK_W)], idbuf)

    def gather(c):
        return pltpu.make_async_copy(
            word_hbm.at[idbuf.at[pl.ds(c * GCHUNK, GCHUNK)]],
            slots.at[lax.rem(c, NSLOT)], gsem)

    def writeback(c):
        return pltpu.make_async_copy(
            slots.at[lax.rem(c, NSLOT)],
            g_hbm.at[pl.ds(w0 + c * GCHUNK, GCHUNK)], wbsem)

    def step(c, carry):
        @pl.when(c < NG)
        def _():
            @pl.when(c >= NSLOT)
            def _():
                writeback(c - NSLOT).wait()
            gather(c).start()

        @pl.when(jnp.logical_and(c >= 2, c - 2 < NG))
        def _():
            gather(c - 2).wait()
            writeback(c - 2).start()

        return carry

    lax.fori_loop(0, NG + 2, step, 0)
    for k in range(NSLOT):
        writeback(NG - NSLOT + k).wait()


def _ln_body(tt_ref, g_ref, pos_ref, type_ref, gam_ref, bet_ref, out_ref):
    x = g_ref[...].astype(jnp.float32)
    tt = tt_ref[...]
    tsel = jnp.where(tt == 1, type_ref[1], type_ref[0])
    x = x + pos_ref[...][None, :, :] + tsel
    mean = jnp.mean(x, axis=-1, keepdims=True)
    xc = x - mean
    var = jnp.mean(xc * xc, axis=-1, keepdims=True)
    y = xc * lax.rsqrt(var + EPS)
    out_ref[...] = y * gam_ref[...] + bet_ref[...]


def kernel(input_ids, token_type_ids, word_emb, pos_emb, type_emb, ln_gamma,
           ln_beta):
    word_bf = word_emb.astype(jnp.bfloat16)
    word_i32 = lax.bitcast_convert_type(
        word_bf.reshape(VOCAB_ROWS, H // 2, 2), jnp.int32)
    mesh = plsc.VectorSubcoreMesh(core_axis_name="c", subcore_axis_name="s")
    gather = pl.kernel(
        _gather_body,
        out_type=jax.ShapeDtypeStruct((TOK, H // 2), jnp.int32),
        mesh=mesh,
        scratch_types=[
            pltpu.VMEM((TOK_W,), jnp.int32),
            pltpu.VMEM((NSLOT, GCHUNK, H // 2), jnp.int32),
            pltpu.SemaphoreType.DMA,
            pltpu.SemaphoreType.DMA,
        ],
    )
    g32 = gather(input_ids.reshape(TOK), word_i32)
    g = lax.bitcast_convert_type(g32, jnp.bfloat16)

    out = pl.pallas_call(
        _ln_body,
        grid=(B // RB,),
        in_specs=[
            pl.BlockSpec((RB, L, 1), lambda i: (i, 0, 0)),
            pl.BlockSpec((RB, L, H), lambda i: (i, 0, 0)),
            pl.BlockSpec((L, H), lambda i: (0, 0)),
            pl.BlockSpec((2, H), lambda i: (0, 0)),
            pl.BlockSpec((H,), lambda i: (0,)),
            pl.BlockSpec((H,), lambda i: (0,)),
        ],
        out_specs=pl.BlockSpec((RB, L, H), lambda i: (i, 0, 0)),
        out_shape=jax.ShapeDtypeStruct((B, L, H), jnp.float32),
    )(token_type_ids.reshape(B, L, 1), g.reshape(B, L, H), pos_emb[:L],
      type_emb, ln_gamma, ln_beta)
    return out


# speedup vs baseline: 7.1245x; 7.1245x over previous
import jax
import jax.numpy as jnp
from jax import lax
from jax.experimental import pallas as pl
from jax.experimental.pallas import tpu as pltpu
from jax.experimental.pallas import tpu_sc as plsc

B = 1024
L = 200
H = 768
HP = H // 2
VOCAB_ROWS = 30522
EPS = 1e-12

NC = 2
NS = 16
NW = NC * NS
TOK = B * L
TOK_W = TOK // NW
GCHUNK = 64
NG = TOK_W // GCHUNK
NSLOT = 4

RV = 512
RB = 8


def _round_bf16(x):
    u = lax.bitcast_convert_type(x, jnp.int32)
    bias = jnp.int32(0x7FFF) + (lax.shift_right_logical(u, 16) & 1)
    return lax.shift_right_logical(u + bias, 16)


def _pack_body(w_ref, o_ref):
    lo = _round_bf16(w_ref[:, :HP])
    hi = _round_bf16(w_ref[:, HP:])
    o_ref[...] = lax.shift_left(hi, 16) | lo


def _gather_body(ids_hbm, word_hbm, g_hbm, idbuf, slots, gsem, wbsem):
    wid = lax.axis_index("s") * NC + lax.axis_index("c")
    w0 = wid * TOK_W
    pltpu.sync_copy(ids_hbm.at[pl.ds(w0, TOK_W)], idbuf)

    def gather(c):
        return pltpu.make_async_copy(
            word_hbm.at[idbuf.at[pl.ds(c * GCHUNK, GCHUNK)]],
            slots.at[lax.rem(c, NSLOT)], gsem)

    def writeback(c):
        return pltpu.make_async_copy(
            slots.at[lax.rem(c, NSLOT)],
            g_hbm.at[pl.ds(w0 + c * GCHUNK, GCHUNK)], wbsem)

    def step(c, carry):
        @pl.when(c < NG)
        def _():
            @pl.when(c >= NSLOT)
            def _():
                writeback(c - NSLOT).wait()
            gather(c).start()

        @pl.when(jnp.logical_and(c >= 2, c - 2 < NG))
        def _():
            gather(c - 2).wait()
            writeback(c - 2).start()

        return carry

    lax.fori_loop(0, NG + 2, step, 0)
    for k in range(NSLOT):
        writeback(NG - NSLOT + k).wait()


def _unpack_lo(g):
    return lax.bitcast_convert_type(lax.shift_left(g, 16), jnp.float32)


def _unpack_hi(g):
    return lax.bitcast_convert_type(g & jnp.int32(-65536), jnp.float32)


def _ln_body(tt_ref, g_ref, pos_ref, type_ref, gam_ref, bet_ref, out_ref):
    g = g_ref[...]
    tt1 = tt_ref[...] == 1
    xl = _unpack_lo(g) + pos_ref[:, :HP][None] + jnp.where(
        tt1, type_ref[1, :HP], type_ref[0, :HP])
    xh = _unpack_hi(g) + pos_ref[:, HP:][None] + jnp.where(
        tt1, type_ref[1, HP:], type_ref[0, HP:])
    s = (jnp.sum(xl, axis=-1, keepdims=True)
         + jnp.sum(xh, axis=-1, keepdims=True))
    mean = s * (1.0 / H)
    xl = xl - mean
    xh = xh - mean
    q = (jnp.sum(xl * xl, axis=-1, keepdims=True)
         + jnp.sum(xh * xh, axis=-1, keepdims=True))
    r = lax.rsqrt(q * (1.0 / H) + EPS)
    out_ref[:, :, :HP] = xl * r * gam_ref[:HP] + bet_ref[:HP]
    out_ref[:, :, HP:] = xh * r * gam_ref[HP:] + bet_ref[HP:]


def kernel(input_ids, token_type_ids, word_emb, pos_emb, type_emb, ln_gamma,
           ln_beta):
    word_packed = pl.pallas_call(
        _pack_body,
        grid=((VOCAB_ROWS + RV - 1) // RV,),
        in_specs=[pl.BlockSpec((RV, H), lambda i: (i, 0))],
        out_specs=pl.BlockSpec((RV, HP), lambda i: (i, 0)),
        out_shape=jax.ShapeDtypeStruct((VOCAB_ROWS, HP), jnp.int32),
    )(word_emb)

    mesh = plsc.VectorSubcoreMesh(core_axis_name="c", subcore_axis_name="s")
    gather = pl.kernel(
        _gather_body,
        out_type=jax.ShapeDtypeStruct((TOK, HP), jnp.int32),
        mesh=mesh,
        scratch_types=[
            pltpu.VMEM((TOK_W,), jnp.int32),
            pltpu.VMEM((NSLOT, GCHUNK, HP), jnp.int32),
            pltpu.SemaphoreType.DMA,
            pltpu.SemaphoreType.DMA,
        ],
    )
    g = gather(input_ids.reshape(TOK), word_packed)

    out = pl.pallas_call(
        _ln_body,
        grid=(B // RB,),
        in_specs=[
            pl.BlockSpec((RB, L, 1), lambda i: (i, 0, 0)),
            pl.BlockSpec((RB, L, HP), lambda i: (i, 0, 0)),
            pl.BlockSpec((L, H), lambda i: (0, 0)),
            pl.BlockSpec((2, H), lambda i: (0, 0)),
            pl.BlockSpec((H,), lambda i: (0,)),
            pl.BlockSpec((H,), lambda i: (0,)),
        ],
        out_specs=pl.BlockSpec((RB, L, H), lambda i: (i, 0, 0)),
        out_shape=jax.ShapeDtypeStruct((B, L, H), jnp.float32),
    )(token_type_ids.reshape(B, L, 1), g.reshape(B, L, HP), pos_emb[:L],
      type_emb, ln_gamma, ln_beta)
    return out


# speedup vs baseline: 7.6602x; 1.0752x over previous
import jax
import jax.numpy as jnp
from jax import lax
from jax.experimental import pallas as pl
from jax.experimental.pallas import tpu as pltpu
from jax.experimental.pallas import tpu_sc as plsc

B = 1024
L = 200
H = 768
HP = H // 2
VOCAB_ROWS = 30522
EPS = 1e-12

NC = 2
NS = 16
NW = NC * NS
TOK = B * L

CH = 2
B_CH = B // CH
TOK_CH = TOK // CH
TOK_W = TOK_CH // NW
GCHUNK = 80
NG = TOK_W // GCHUNK
NSLOT = 4

RV = 1024
RB = 16


def _round_bf16(x):
    u = lax.bitcast_convert_type(x, jnp.int32)
    bias = jnp.int32(0x7FFF) + (lax.shift_right_logical(u, 16) & 1)
    return lax.shift_right_logical(u + bias, 16)


def _pack_body(w_ref, o_ref):
    lo = _round_bf16(w_ref[:, :HP])
    hi = _round_bf16(w_ref[:, HP:])
    o_ref[...] = lax.shift_left(hi, 16) | lo


def _gather_body(ids_hbm, word_hbm, g_hbm, idbuf, slots, gsem, wbsem):
    wid = lax.axis_index("s") * NC + lax.axis_index("c")
    w0 = wid * TOK_W
    pltpu.sync_copy(ids_hbm.at[pl.ds(w0, TOK_W)], idbuf)

    def gather(c):
        return pltpu.make_async_copy(
            word_hbm.at[idbuf.at[pl.ds(c * GCHUNK, GCHUNK)]],
            slots.at[lax.rem(c, NSLOT)], gsem)

    def writeback(c):
        return pltpu.make_async_copy(
            slots.at[lax.rem(c, NSLOT)],
            g_hbm.at[pl.ds(w0 + c * GCHUNK, GCHUNK)], wbsem)

    def step(c, carry):
        @pl.when(c < NG)
        def _():
            @pl.when(c >= NSLOT)
            def _():
                writeback(c - NSLOT).wait()
            gather(c).start()

        @pl.when(jnp.logical_and(c >= 2, c - 2 < NG))
        def _():
            gather(c - 2).wait()
            writeback(c - 2).start()

        return carry

    lax.fori_loop(0, NG + 2, step, 0)
    for k in range(NSLOT):
        writeback(NG - NSLOT + k).wait()


def _unpack_lo(g):
    return lax.bitcast_convert_type(lax.shift_left(g, 16), jnp.float32)


def _unpack_hi(g):
    return lax.bitcast_convert_type(g & jnp.int32(-65536), jnp.float32)


def _ln_math(tt_ref, g_ref, pos_ref, type_ref, gam_ref, bet_ref, out_ref):
    g = g_ref[...]
    tt1 = tt_ref[...] == 1
    xl = _unpack_lo(g) + pos_ref[:, :HP][None] + jnp.where(
        tt1, type_ref[1, :HP], type_ref[0, :HP])
    xh = _unpack_hi(g) + pos_ref[:, HP:][None] + jnp.where(
        tt1, type_ref[1, HP:], type_ref[0, HP:])
    s = (jnp.sum(xl, axis=-1, keepdims=True)
         + jnp.sum(xh, axis=-1, keepdims=True))
    mean = s * (1.0 / H)
    xl = xl - mean
    xh = xh - mean
    q = (jnp.sum(xl * xl, axis=-1, keepdims=True)
         + jnp.sum(xh * xh, axis=-1, keepdims=True))
    r = lax.rsqrt(q * (1.0 / H) + EPS)
    out_ref[:, :, :HP] = xl * r * gam_ref[:HP] + bet_ref[:HP]
    out_ref[:, :, HP:] = xh * r * gam_ref[HP:] + bet_ref[HP:]


def _ln_body_first(tt_ref, g_ref, pos_ref, type_ref, gam_ref, bet_ref,
                   out_ref):
    _ln_math(tt_ref, g_ref, pos_ref, type_ref, gam_ref, bet_ref, out_ref)


def _ln_body_alias(full_ref, tt_ref, g_ref, pos_ref, type_ref, gam_ref,
                   bet_ref, out_ref):
    del full_ref
    _ln_math(tt_ref, g_ref, pos_ref, type_ref, gam_ref, bet_ref, out_ref)


def kernel(input_ids, token_type_ids, word_emb, pos_emb, type_emb, ln_gamma,
           ln_beta):
    word_packed = pl.pallas_call(
        _pack_body,
        grid=((VOCAB_ROWS + RV - 1) // RV,),
        in_specs=[pl.BlockSpec((RV, H), lambda i: (i, 0))],
        out_specs=pl.BlockSpec((RV, HP), lambda i: (i, 0)),
        out_shape=jax.ShapeDtypeStruct((VOCAB_ROWS, HP), jnp.int32),
    )(word_emb)

    mesh = plsc.VectorSubcoreMesh(core_axis_name="c", subcore_axis_name="s")
    gather = pl.kernel(
        _gather_body,
        out_type=jax.ShapeDtypeStruct((TOK_CH, HP), jnp.int32),
        mesh=mesh,
        scratch_types=[
            pltpu.VMEM((TOK_W,), jnp.int32),
            pltpu.VMEM((NSLOT, GCHUNK, HP), jnp.int32),
            pltpu.SemaphoreType.DMA,
            pltpu.SemaphoreType.DMA,
        ],
    )
    ids_flat = input_ids.reshape(TOK)
    gs = [gather(lax.slice(ids_flat, (k * TOK_CH,), ((k + 1) * TOK_CH,)),
                 word_packed)
          for k in range(CH)]

    tt3 = token_type_ids.reshape(B, L, 1)
    nblk = B_CH // RB
    common_specs = [
        pl.BlockSpec((L, H), lambda i: (0, 0)),
        pl.BlockSpec((2, H), lambda i: (0, 0)),
        pl.BlockSpec((H,), lambda i: (0,)),
        pl.BlockSpec((H,), lambda i: (0,)),
    ]
    out = None
    for k in range(CH):
        off = k * nblk
        tt_k = lax.slice(tt3, (k * B_CH, 0, 0), ((k + 1) * B_CH, L, 1))
        g_k = gs[k].reshape(B_CH, L, HP)
        chunk_specs = [
            pl.BlockSpec((RB, L, 1), lambda i: (i, 0, 0)),
            pl.BlockSpec((RB, L, HP), lambda i: (i, 0, 0)),
        ]
        out_spec = pl.BlockSpec((RB, L, H),
                                lambda i, o=off: (i + o, 0, 0))
        if k == 0:
            out = pl.pallas_call(
                _ln_body_first,
                grid=(nblk,),
                in_specs=chunk_specs + common_specs,
                out_specs=out_spec,
                out_shape=jax.ShapeDtypeStruct((B, L, H), jnp.float32),
            )(tt_k, g_k, pos_emb[:L], type_emb, ln_gamma, ln_beta)
        else:
            out = pl.pallas_call(
                _ln_body_alias,
                grid=(nblk,),
                in_specs=[pl.BlockSpec(memory_space=pltpu.MemorySpace.HBM)]
                + chunk_specs + common_specs,
                out_specs=out_spec,
                out_shape=jax.ShapeDtypeStruct((B, L, H), jnp.float32),
                input_output_aliases={0: 0},
            )(out, tt_k, g_k, pos_emb[:L], type_emb, ln_gamma, ln_beta)
    return out


# speedup vs baseline: 7.6833x; 1.0030x over previous
import jax
import jax.numpy as jnp
from jax import lax
from jax.experimental import pallas as pl
from jax.experimental.pallas import tpu as pltpu
from jax.experimental.pallas import tpu_sc as plsc

B = 1024
L = 200
H = 768
HP = H // 2
VOCAB_ROWS = 30522
EPS = 1e-12

NC = 2
NS = 16
NW = NC * NS
TOK = B * L

CH = 4
B_CH = B // CH
TOK_CH = TOK // CH
TOK_W = TOK_CH // NW
GCHUNK = 80
NG = TOK_W // GCHUNK
NSLOT = 4

RV = 1024
RB = 16


def _round_bf16(x):
    u = lax.bitcast_convert_type(x, jnp.int32)
    bias = jnp.int32(0x7FFF) + (lax.shift_right_logical(u, 16) & 1)
    return lax.shift_right_logical(u + bias, 16)


def _pack_body(w_ref, o_ref):
    lo = _round_bf16(w_ref[:, :HP])
    hi = _round_bf16(w_ref[:, HP:])
    o_ref[...] = lax.shift_left(hi, 16) | lo


def _gather_body(ids_hbm, word_hbm, g_hbm, idbuf, slots, gsem, wbsem):
    wid = lax.axis_index("s") * NC + lax.axis_index("c")
    w0 = wid * TOK_W
    pltpu.sync_copy(ids_hbm.at[pl.ds(w0, TOK_W)], idbuf)

    def gather(c):
        return pltpu.make_async_copy(
            word_hbm.at[idbuf.at[pl.ds(c * GCHUNK, GCHUNK)]],
            slots.at[lax.rem(c, NSLOT)], gsem)

    def writeback(c):
        return pltpu.make_async_copy(
            slots.at[lax.rem(c, NSLOT)],
            g_hbm.at[pl.ds(w0 + c * GCHUNK, GCHUNK)], wbsem)

    def step(c, carry):
        @pl.when(c < NG)
        def _():
            @pl.when(c >= NSLOT)
            def _():
                writeback(c - NSLOT).wait()
            gather(c).start()

        @pl.when(jnp.logical_and(c >= 2, c - 2 < NG))
        def _():
            gather(c - 2).wait()
            writeback(c - 2).start()

        return carry

    lax.fori_loop(0, NG + 2, step, 0)
    for k in range(NSLOT):
        writeback(NG - NSLOT + k).wait()


def _unpack_lo(g):
    return lax.bitcast_convert_type(lax.shift_left(g, 16), jnp.float32)


def _unpack_hi(g):
    return lax.bitcast_convert_type(g & jnp.int32(-65536), jnp.float32)


def _ln_math(tt_ref, g_ref, pos_ref, type_ref, gam_ref, bet_ref, out_ref):
    g = g_ref[...]
    tt1 = tt_ref[...] == 1
    xl = _unpack_lo(g) + pos_ref[:, :HP][None] + jnp.where(
        tt1, type_ref[1, :HP], type_ref[0, :HP])
    xh = _unpack_hi(g) + pos_ref[:, HP:][None] + jnp.where(
        tt1, type_ref[1, HP:], type_ref[0, HP:])
    s = (jnp.sum(xl, axis=-1, keepdims=True)
         + jnp.sum(xh, axis=-1, keepdims=True))
    mean = s * (1.0 / H)
    xl = xl - mean
    xh = xh - mean
    q = (jnp.sum(xl * xl, axis=-1, keepdims=True)
         + jnp.sum(xh * xh, axis=-1, keepdims=True))
    r = lax.rsqrt(q * (1.0 / H) + EPS)
    out_ref[:, :, :HP] = xl * r * gam_ref[:HP] + bet_ref[:HP]
    out_ref[:, :, HP:] = xh * r * gam_ref[HP:] + bet_ref[HP:]


def _ln_body_first(tt_ref, g_ref, pos_ref, type_ref, gam_ref, bet_ref,
                   out_ref):
    _ln_math(tt_ref, g_ref, pos_ref, type_ref, gam_ref, bet_ref, out_ref)


def _ln_body_alias(full_ref, tt_ref, g_ref, pos_ref, type_ref, gam_ref,
                   bet_ref, out_ref):
    del full_ref
    _ln_math(tt_ref, g_ref, pos_ref, type_ref, gam_ref, bet_ref, out_ref)


def kernel(input_ids, token_type_ids, word_emb, pos_emb, type_emb, ln_gamma,
           ln_beta):
    word_packed = pl.pallas_call(
        _pack_body,
        grid=((VOCAB_ROWS + RV - 1) // RV,),
        in_specs=[pl.BlockSpec((RV, H), lambda i: (i, 0))],
        out_specs=pl.BlockSpec((RV, HP), lambda i: (i, 0)),
        out_shape=jax.ShapeDtypeStruct((VOCAB_ROWS, HP), jnp.int32),
    )(word_emb)

    mesh = plsc.VectorSubcoreMesh(core_axis_name="c", subcore_axis_name="s")
    gather = pl.kernel(
        _gather_body,
        out_type=jax.ShapeDtypeStruct((TOK_CH, HP), jnp.int32),
        mesh=mesh,
        scratch_types=[
            pltpu.VMEM((TOK_W,), jnp.int32),
            pltpu.VMEM((NSLOT, GCHUNK, HP), jnp.int32),
            pltpu.SemaphoreType.DMA,
            pltpu.SemaphoreType.DMA,
        ],
    )
    ids_flat = input_ids.reshape(TOK)
    gs = [gather(lax.slice(ids_flat, (k * TOK_CH,), ((k + 1) * TOK_CH,)),
                 word_packed)
          for k in range(CH)]

    tt3 = token_type_ids.reshape(B, L, 1)
    nblk = B_CH // RB
    common_specs = [
        pl.BlockSpec((L, H), lambda i: (0, 0)),
        pl.BlockSpec((2, H), lambda i: (0, 0)),
        pl.BlockSpec((H,), lambda i: (0,)),
        pl.BlockSpec((H,), lambda i: (0,)),
    ]
    out = None
    for k in range(CH):
        off = k * nblk
        tt_k = lax.slice(tt3, (k * B_CH, 0, 0), ((k + 1) * B_CH, L, 1))
        g_k = gs[k].reshape(B_CH, L, HP)
        chunk_specs = [
            pl.BlockSpec((RB, L, 1), lambda i: (i, 0, 0)),
            pl.BlockSpec((RB, L, HP), lambda i: (i, 0, 0)),
        ]
        out_spec = pl.BlockSpec((RB, L, H),
                                lambda i, o=off: (i + o, 0, 0))
        if k == 0:
            out = pl.pallas_call(
                _ln_body_first,
                grid=(nblk,),
                in_specs=chunk_specs + common_specs,
                out_specs=out_spec,
                out_shape=jax.ShapeDtypeStruct((B, L, H), jnp.float32),
            )(tt_k, g_k, pos_emb[:L], type_emb, ln_gamma, ln_beta)
        else:
            out = pl.pallas_call(
                _ln_body_alias,
                grid=(nblk,),
                in_specs=[pl.BlockSpec(memory_space=pltpu.MemorySpace.HBM)]
                + chunk_specs + common_specs,
                out_specs=out_spec,
                out_shape=jax.ShapeDtypeStruct((B, L, H), jnp.float32),
                input_output_aliases={0: 0},
            )(out, tt_k, g_k, pos_emb[:L], type_emb, ln_gamma, ln_beta)
    return out
